# baseline (speedup 1.0000x reference)
import functools

import jax
import jax.numpy as jnp
from jax.experimental import pallas as pl
from jax.experimental.pallas import tpu as pltpu
from jax.experimental.pallas import tpu_sc as plsc

S, D, H, OUTD = 2048, 768, 3072, 768
G, EG = 4, 4
E = G * EG
BT = 256
NBCAP = (2 * S) // BT + E
PCAP = NBCAP * BT

_BF = jnp.bfloat16


def _router_body(x_ref, wgT_ref, bg_ref, werT_ref, ber_ref, eid_ref, w_ref):
    xb = x_ref[...]
    gl = jax.lax.dot_general(wgT_ref[...], xb, (((1,), (1,)), ((), ())),
                             preferred_element_type=jnp.float32) \
        + bg_ref[...]
    ridx = jax.lax.broadcasted_iota(jnp.int32, (G, S), 0)
    big = jnp.int32(G + 1)

    m = jnp.max(gl, axis=0, keepdims=True)
    egl = jnp.exp(gl - m)
    gp = egl / jnp.sum(egl, axis=0, keepdims=True)
    v1 = jnp.max(gp, axis=0, keepdims=True)
    i1 = jnp.min(jnp.where(gp == v1, ridx, big), axis=0, keepdims=True)
    gp2 = jnp.where(ridx == i1, -1.0, gp)
    v2 = jnp.max(gp2, axis=0, keepdims=True)
    i2 = jnp.min(jnp.where(gp2 == v2, ridx, big), axis=0, keepdims=True)

    ew = []
    ei = []
    for g in range(G):
        el = jax.lax.dot_general(werT_ref[g], xb, (((1,), (1,)), ((), ())),
                                 preferred_element_type=jnp.float32) \
            + ber_ref[g]
        mg = jnp.max(el, axis=0, keepdims=True)
        ei.append(jnp.min(jnp.where(el == mg, ridx, big), axis=0, keepdims=True))
        ew.append(1.0 / jnp.sum(jnp.exp(el - mg), axis=0, keepdims=True))

    rows_eid, rows_w = [], []
    for gsel, gwk in ((i1, v1), (i2, v2)):
        ew_sel = jnp.zeros((1, S), jnp.float32)
        ei_sel = jnp.zeros((1, S), jnp.int32)
        for g in range(G):
            hit = gsel == g
            ew_sel = jnp.where(hit, ew[g], ew_sel)
            ei_sel = jnp.where(hit, ei[g], ei_sel)
        rows_eid.append(gsel * EG + ei_sel)
        rows_w.append(gwk * ew_sel)
    eid_ref[...] = jnp.concatenate(rows_eid, axis=0)
    w_ref[...] = jnp.concatenate(rows_w, axis=0)


def _route(x_bf, Wg, bg, Wer, ber):
    wgT = Wg.T.astype(_BF)
    bg2 = bg.reshape(G, 1)
    werT = Wer.transpose(0, 2, 1).astype(_BF)
    ber3 = ber.reshape(G, EG, 1)
    return pl.pallas_call(
        _router_body,
        out_shape=(jax.ShapeDtypeStruct((2, S), jnp.int32),
                   jax.ShapeDtypeStruct((2, S), jnp.float32)),
    )(x_bf, wgT, bg2, werT, ber3)


def _gather_rows(table, idx):
    n = idx.shape[0]
    dm = table.shape[1]
    hdm = dm // 2
    win = 128
    table2 = table.reshape(2 * table.shape[0], hdm)
    idx2 = jnp.stack([2 * idx, 2 * idx + 1], axis=-1).reshape(1, 2 * n)
    mesh = plsc.VectorSubcoreMesh(core_axis_name="c", subcore_axis_name="s")

    @functools.partial(pl.kernel,
                       out_type=jax.ShapeDtypeStruct((2 * n, hdm), table.dtype),
                       mesh=mesh)
    def k(x_hbm, i_hbm, o_hbm):
        def body(i_vmem, o_vmem):
            pltpu.sync_copy(x_hbm.at[i_vmem.at[0]], o_vmem)

        pltpu.emit_pipeline(
            body,
            grid=(2 * n // win,),
            in_specs=[pl.BlockSpec((1, win), lambda i: (0, i))],
            out_specs=[pl.BlockSpec((win, hdm), lambda i: (i, 0))],
            core_axis_name=("c", "s"),
            dimension_semantics=(pltpu.PARALLEL,),
        )(i_hbm, o_hbm)

    return k(table2, idx2).reshape(n, dm)


def _gather_scatter(table, src_idx, dst_idx):
    n = src_idx.shape[0]
    dm = table.shape[1]
    hdm = dm // 2
    win = 128
    table2 = table.reshape(2 * table.shape[0], hdm)
    si2 = jnp.stack([2 * src_idx, 2 * src_idx + 1], axis=-1).reshape(1, 2 * n)
    di2 = jnp.stack([2 * dst_idx, 2 * dst_idx + 1], axis=-1).reshape(1, 2 * n)
    mesh = plsc.VectorSubcoreMesh(core_axis_name="c", subcore_axis_name="s")

    @functools.partial(pl.kernel,
                       out_type=jax.ShapeDtypeStruct((2 * n, hdm), table.dtype),
                       mesh=mesh,
                       scratch_types=[pltpu.VMEM((win, hdm), table.dtype)])
    def k(x_hbm, si_hbm, di_hbm, o_hbm, rows_v):
        def body(si_vmem, di_vmem):
            pltpu.sync_copy(x_hbm.at[si_vmem.at[0]], rows_v)
            pltpu.sync_copy(rows_v, o_hbm.at[di_vmem.at[0]])

        pltpu.emit_pipeline(
            body,
            grid=(2 * n // win,),
            in_specs=[pl.BlockSpec((1, win), lambda i: (0, i)),
                      pl.BlockSpec((1, win), lambda i: (0, i))],
            out_specs=[],
            core_axis_name=("c", "s"),
            dimension_semantics=(pltpu.PARALLEL,),
        )(si_hbm, di_hbm)

    return k(table2, si2, di2).reshape(n, dm)


def _ffn_body(be_ref, nu_ref, tok_ref, rw_ref, x_ref, w1_ref, b1_ref, w2_ref,
              b2_ref, out_ref):
    b = pl.program_id(0)

    @pl.when(b < nu_ref[0])
    def _():
        tid = tok_ref[0]
        lanes = jax.lax.broadcasted_iota(jnp.int32, (BT, S), 1)
        oh = (lanes == tid).astype(_BF)
        xv = jnp.dot(oh, x_ref[...],
                     preferred_element_type=jnp.float32).astype(_BF)
        h = jnp.dot(xv, w1_ref[0].astype(_BF),
                    preferred_element_type=jnp.float32) + b1_ref[0]
        h = jax.nn.gelu(h.astype(_BF))
        y = jnp.dot(h, w2_ref[0].astype(_BF),
                    preferred_element_type=jnp.float32) + b2_ref[0]
        out_ref[...] = y * rw_ref[0]


def _ffn(x_bf, tok3, rw3, block_expert, nb_used, W1, b1, W2, b2):
    w1r = W1.reshape(E, D, H)
    b1r = b1.reshape(E, 1, H)
    w2r = W2.reshape(E, H, OUTD)
    b2r = b2.reshape(E, 1, OUTD)
    grid_spec = pltpu.PrefetchScalarGridSpec(
        num_scalar_prefetch=2,
        grid=(NBCAP,),
        in_specs=[
            pl.BlockSpec((1, BT, 1), lambda b, be, nu: (b, 0, 0)),
            pl.BlockSpec((1, BT, 1), lambda b, be, nu: (b, 0, 0)),
            pl.BlockSpec((S, D), lambda b, be, nu: (0, 0)),
            pl.BlockSpec((1, D, H), lambda b, be, nu: (be[b], 0, 0)),
            pl.BlockSpec((1, 1, H), lambda b, be, nu: (be[b], 0, 0)),
            pl.BlockSpec((1, H, OUTD), lambda b, be, nu: (be[b], 0, 0)),
            pl.BlockSpec((1, 1, OUTD), lambda b, be, nu: (be[b], 0, 0)),
        ],
        out_specs=pl.BlockSpec((BT, OUTD), lambda b, be, nu: (b, 0)),
    )
    return pl.pallas_call(
        _ffn_body,
        grid_spec=grid_spec,
        out_shape=jax.ShapeDtypeStruct((PCAP, OUTD), jnp.float32),
    )(block_expert, nb_used, tok3, rw3, x_bf, w1r, b1r, w2r, b2r)


def _combine_body(yg_ref, wo_ref, bo_ref, gam_ref, bet_ref, out_ref):
    comb = yg_ref[0:S, :] + yg_ref[S:2 * S, :]
    z = jnp.dot(comb.astype(_BF), wo_ref[...].astype(_BF),
                preferred_element_type=jnp.float32) + bo_ref[...]
    mu = jnp.mean(z, axis=-1, keepdims=True)
    var = jnp.mean((z - mu) ** 2, axis=-1, keepdims=True)
    out_ref[...] = (z - mu) * jax.lax.rsqrt(var + 1e-5) * gam_ref[...] \
        + bet_ref[...]


def _combine(yg, Wo, bo, gamma, beta):
    return pl.pallas_call(
        _combine_body,
        out_shape=jax.ShapeDtypeStruct((S, OUTD), jnp.float32),
    )(yg, Wo, bo.reshape(1, OUTD),
      gamma.reshape(1, OUTD), beta.reshape(1, OUTD))


def kernel(x, Wg, bg, Wer, ber, W1, b1, W2, b2, Wo, bo, gamma, beta):
    x2 = x.reshape(S, D)
    x_bf2 = x2.astype(_BF)
    eid, w = _route(x_bf2, Wg, bg, Wer, ber)

    eid_flat = eid.reshape(-1)
    order = jnp.argsort(eid_flat, stable=True).astype(jnp.int32)
    sorted_eid = eid_flat[order]
    w_sorted = w.reshape(-1)[order]
    tok_sorted = (order % S).astype(jnp.int32)
    ear = jnp.arange(E, dtype=jnp.int32)
    offs = jnp.searchsorted(sorted_eid, ear, side="left").astype(jnp.int32)
    cend = jnp.searchsorted(sorted_eid, ear, side="right").astype(jnp.int32)
    counts = cend - offs
    pc = ((counts + BT - 1) // BT) * BT
    pstart = jnp.cumsum(pc) - pc
    jj = jnp.arange(2 * S, dtype=jnp.int32)
    ppos = (pstart[sorted_eid] + jj - offs[sorted_eid]).astype(jnp.int32)
    block_expert = (jnp.searchsorted(pstart // BT, jnp.arange(NBCAP),
                                     side="right") - 1).astype(jnp.int32)
    nb_used = ((pstart[E - 1] + pc[E - 1]) // BT).astype(jnp.int32).reshape(1)
    e_of_p = jnp.repeat(block_expert, BT)
    rel = jnp.arange(PCAP, dtype=jnp.int32) - pstart[e_of_p]
    jv = jnp.minimum(offs[e_of_p] + rel, 2 * S - 1)
    valid = rel < counts[e_of_p]
    tok_padded = tok_sorted[jv]
    rw_padded = jnp.where(valid, w_sorted[jv], 0.0)

    ys = _ffn(x_bf2, tok_padded.reshape(NBCAP, BT, 1),
              rw_padded.reshape(NBCAP, BT, 1), block_expert, nb_used,
              W1, b1, W2, b2)
    yg = _gather_scatter(ys, ppos, order)
    out = _combine(yg, Wo, bo, gamma, beta)
    return out.reshape(1, S, OUTD)


# speedup vs baseline: 1.5847x; 1.5847x over previous
import functools

import jax
import jax.numpy as jnp
from jax.experimental import pallas as pl
from jax.experimental.pallas import tpu as pltpu
from jax.experimental.pallas import tpu_sc as plsc

S, D, H, OUTD = 2048, 768, 3072, 768
G, EG = 4, 4
E = G * EG
BT = 256
NBCAP = (2 * S) // BT + E
PCAP = NBCAP * BT

_BF = jnp.bfloat16


def _router_body(x_ref, wgT_ref, bg_ref, werT_ref, ber_ref, eid_ref, w_ref):
    xb = x_ref[...]
    gl = jax.lax.dot_general(wgT_ref[...], xb, (((1,), (1,)), ((), ())),
                             preferred_element_type=jnp.float32) \
        + bg_ref[...]
    ridx = jax.lax.broadcasted_iota(jnp.int32, (G, S), 0)
    big = jnp.int32(G + 1)

    m = jnp.max(gl, axis=0, keepdims=True)
    egl = jnp.exp(gl - m)
    gp = egl / jnp.sum(egl, axis=0, keepdims=True)
    v1 = jnp.max(gp, axis=0, keepdims=True)
    i1 = jnp.min(jnp.where(gp == v1, ridx, big), axis=0, keepdims=True)
    gp2 = jnp.where(ridx == i1, -1.0, gp)
    v2 = jnp.max(gp2, axis=0, keepdims=True)
    i2 = jnp.min(jnp.where(gp2 == v2, ridx, big), axis=0, keepdims=True)

    ew = []
    ei = []
    for g in range(G):
        el = jax.lax.dot_general(werT_ref[g], xb, (((1,), (1,)), ((), ())),
                                 preferred_element_type=jnp.float32) \
            + ber_ref[g]
        mg = jnp.max(el, axis=0, keepdims=True)
        ei.append(jnp.min(jnp.where(el == mg, ridx, big), axis=0, keepdims=True))
        ew.append(1.0 / jnp.sum(jnp.exp(el - mg), axis=0, keepdims=True))

    rows_eid, rows_w = [], []
    for gsel, gwk in ((i1, v1), (i2, v2)):
        ew_sel = jnp.zeros((1, S), jnp.float32)
        ei_sel = jnp.zeros((1, S), jnp.int32)
        for g in range(G):
            hit = gsel == g
            ew_sel = jnp.where(hit, ew[g], ew_sel)
            ei_sel = jnp.where(hit, ei[g], ei_sel)
        rows_eid.append(gsel * EG + ei_sel)
        rows_w.append(gwk * ew_sel)
    eid_ref[...] = jnp.concatenate(rows_eid, axis=0)
    w_ref[...] = jnp.concatenate(rows_w, axis=0)


def _route(x_bf, Wg, bg, Wer, ber):
    wgT = Wg.T.astype(_BF)
    bg2 = bg.reshape(G, 1)
    werT = Wer.transpose(0, 2, 1).astype(_BF)
    ber3 = ber.reshape(G, EG, 1)
    return pl.pallas_call(
        _router_body,
        out_shape=(jax.ShapeDtypeStruct((2, S), jnp.int32),
                   jax.ShapeDtypeStruct((2, S), jnp.float32)),
    )(x_bf, wgT, bg2, werT, ber3)


def _gather_rows(table, idx):
    n = idx.shape[0]
    dm = table.shape[1]
    hdm = dm // 2
    win = 128
    table2 = table.reshape(2 * table.shape[0], hdm)
    idx2 = jnp.stack([2 * idx, 2 * idx + 1], axis=-1).reshape(1, 2 * n)
    mesh = plsc.VectorSubcoreMesh(core_axis_name="c", subcore_axis_name="s")

    @functools.partial(pl.kernel,
                       out_type=jax.ShapeDtypeStruct((2 * n, hdm), table.dtype),
                       mesh=mesh)
    def k(x_hbm, i_hbm, o_hbm):
        def body(i_vmem, o_vmem):
            pltpu.sync_copy(x_hbm.at[i_vmem.at[0]], o_vmem)

        pltpu.emit_pipeline(
            body,
            grid=(2 * n // win,),
            in_specs=[pl.BlockSpec((1, win), lambda i: (0, i))],
            out_specs=[pl.BlockSpec((win, hdm), lambda i: (i, 0))],
            core_axis_name=("c", "s"),
            dimension_semantics=(pltpu.PARALLEL,),
        )(i_hbm, o_hbm)

    return k(table2, idx2).reshape(n, dm)


def _gather_scatter(table, src_idx, dst_idx):
    n = src_idx.shape[0]
    dm = table.shape[1]
    hdm = dm // 2
    win = 128
    table2 = table.reshape(2 * table.shape[0], hdm)
    si2 = jnp.stack([2 * src_idx, 2 * src_idx + 1], axis=-1).reshape(1, 2 * n)
    di2 = jnp.stack([2 * dst_idx, 2 * dst_idx + 1], axis=-1).reshape(1, 2 * n)
    mesh = plsc.VectorSubcoreMesh(core_axis_name="c", subcore_axis_name="s")

    @functools.partial(pl.kernel,
                       out_type=jax.ShapeDtypeStruct((2 * n, hdm), table.dtype),
                       mesh=mesh,
                       scratch_types=[pltpu.VMEM((win, hdm), table.dtype)])
    def k(x_hbm, si_hbm, di_hbm, o_hbm, rows_v):
        def body(si_vmem, di_vmem):
            pltpu.sync_copy(x_hbm.at[si_vmem.at[0]], rows_v)
            pltpu.sync_copy(rows_v, o_hbm.at[di_vmem.at[0]])

        pltpu.emit_pipeline(
            body,
            grid=(2 * n // win,),
            in_specs=[pl.BlockSpec((1, win), lambda i: (0, i)),
                      pl.BlockSpec((1, win), lambda i: (0, i))],
            out_specs=[],
            core_axis_name=("c", "s"),
            dimension_semantics=(pltpu.PARALLEL,),
        )(si_hbm, di_hbm)

    return k(table2, si2, di2).reshape(n, dm)


def _ffn_body(be_ref, nu_ref, tok_ref, x_ref, w1_ref, b1_ref, w2_ref,
              b2_ref, out_ref):
    b = pl.program_id(0)

    @pl.when(b < nu_ref[0])
    def _():
        tid = tok_ref[0]
        rows = jax.lax.broadcasted_iota(jnp.int32, (S, BT), 0)
        ohT = (rows == tid).astype(_BF)
        xv = jax.lax.dot_general(ohT, x_ref[...], (((0,), (0,)), ((), ())),
                                 preferred_element_type=jnp.float32
                                 ).astype(_BF)
        h = jnp.dot(xv, w1_ref[0].astype(_BF),
                    preferred_element_type=jnp.float32) + b1_ref[0]
        h = jax.nn.gelu(h.astype(_BF))
        out_ref[...] = jnp.dot(h, w2_ref[0].astype(_BF),
                               preferred_element_type=jnp.float32) + b2_ref[0]


def _ffn(x_bf, tok3, block_expert, nb_used, W1, b1, W2, b2):
    w1r = W1.reshape(E, D, H)
    b1r = b1.reshape(E, 1, H)
    w2r = W2.reshape(E, H, OUTD)
    b2r = b2.reshape(E, 1, OUTD)
    grid_spec = pltpu.PrefetchScalarGridSpec(
        num_scalar_prefetch=2,
        grid=(NBCAP,),
        in_specs=[
            pl.BlockSpec((1, 1, BT), lambda b, be, nu: (b, 0, 0)),
            pl.BlockSpec((S, D), lambda b, be, nu: (0, 0)),
            pl.BlockSpec((1, D, H), lambda b, be, nu: (be[b], 0, 0)),
            pl.BlockSpec((1, 1, H), lambda b, be, nu: (be[b], 0, 0)),
            pl.BlockSpec((1, H, OUTD), lambda b, be, nu: (be[b], 0, 0)),
            pl.BlockSpec((1, 1, OUTD), lambda b, be, nu: (be[b], 0, 0)),
        ],
        out_specs=pl.BlockSpec((BT, OUTD), lambda b, be, nu: (b, 0)),
    )
    return pl.pallas_call(
        _ffn_body,
        grid_spec=grid_spec,
        out_shape=jax.ShapeDtypeStruct((PCAP, OUTD), jnp.float32),
    )(block_expert, nb_used, tok3, x_bf, w1r, b1r, w2r, b2r)


def _combine_body(yg_ref, w0_ref, w1_ref, wo_ref, bo_ref, gam_ref, bet_ref,
                  out_ref):
    comb = (w0_ref[...] * yg_ref[0:S, :] + w1_ref[...] * yg_ref[S:2 * S, :])
    z = jnp.dot(comb.astype(_BF), wo_ref[...].astype(_BF),
                preferred_element_type=jnp.float32) + bo_ref[...]
    mu = jnp.mean(z, axis=-1, keepdims=True)
    var = jnp.mean((z - mu) ** 2, axis=-1, keepdims=True)
    out_ref[...] = (z - mu) * jax.lax.rsqrt(var + 1e-5) * gam_ref[...] \
        + bet_ref[...]


def _combine(yg, w0c, w1c, Wo, bo, gamma, beta):
    return pl.pallas_call(
        _combine_body,
        out_shape=jax.ShapeDtypeStruct((S, OUTD), jnp.float32),
    )(yg, w0c, w1c, Wo, bo.reshape(1, OUTD),
      gamma.reshape(1, OUTD), beta.reshape(1, OUTD))


def kernel(x, Wg, bg, Wer, ber, W1, b1, W2, b2, Wo, bo, gamma, beta):
    x2 = x.reshape(S, D)
    x_bf2 = x2.astype(_BF)
    eid, w = _route(x_bf2, Wg, bg, Wer, ber)

    eid_flat = eid.reshape(-1)
    order = jnp.argsort(eid_flat, stable=True).astype(jnp.int32)
    sorted_eid = eid_flat[order]
    tok_sorted = (order % S).astype(jnp.int32)
    counts = jnp.bincount(eid_flat, length=E)
    offs = jnp.cumsum(counts) - counts
    pc = ((counts + BT - 1) // BT) * BT
    pstart = jnp.cumsum(pc) - pc
    jj = jnp.arange(2 * S, dtype=jnp.int32)
    ppos = (pstart[sorted_eid] + jj - offs[sorted_eid]).astype(jnp.int32)
    tok_padded = jnp.zeros((PCAP,), jnp.int32).at[ppos].set(tok_sorted)
    block_expert = (jnp.searchsorted(pstart // BT, jnp.arange(NBCAP),
                                     side="right") - 1).astype(jnp.int32)
    nb_used = ((pstart[E - 1] + pc[E - 1]) // BT).astype(jnp.int32).reshape(1)

    ys = _ffn(x_bf2, tok_padded.reshape(NBCAP, 1, BT), block_expert, nb_used,
              W1, b1, W2, b2)
    yg = _gather_scatter(ys, ppos, order)
    wt = w.T
    out = _combine(yg, wt[:, 0:1], wt[:, 1:2], Wo, bo, gamma, beta)
    return out.reshape(1, S, OUTD)


# speedup vs baseline: 2.3643x; 1.4919x over previous
import functools

import jax
import jax.numpy as jnp
from jax.experimental import pallas as pl
from jax.experimental.pallas import tpu as pltpu
from jax.experimental.pallas import tpu_sc as plsc

S, D, H, OUTD = 2048, 768, 3072, 768
G, EG = 4, 4
E = G * EG
BT = 256
NBCAP = (2 * S) // BT + E
PCAP = NBCAP * BT

_BF = jnp.bfloat16
_HI = jax.lax.Precision.HIGHEST


def _router_body(x_ref, wgT_ref, bg_ref, werT_ref, ber_ref,
                 ppos_ref, w_ref, meta_ref):
    xb = x_ref[...]
    gl = jax.lax.dot_general(wgT_ref[...], xb, (((1,), (1,)), ((), ())),
                             preferred_element_type=jnp.float32) \
        + bg_ref[...]
    ridx = jax.lax.broadcasted_iota(jnp.int32, (G, S), 0)
    big = jnp.int32(G + 1)

    m = jnp.max(gl, axis=0, keepdims=True)
    egl = jnp.exp(gl - m)
    gp = egl / jnp.sum(egl, axis=0, keepdims=True)
    v1 = jnp.max(gp, axis=0, keepdims=True)
    i1 = jnp.min(jnp.where(gp == v1, ridx, big), axis=0, keepdims=True)
    gp2 = jnp.where(ridx == i1, -1.0, gp)
    v2 = jnp.max(gp2, axis=0, keepdims=True)
    i2 = jnp.min(jnp.where(gp2 == v2, ridx, big), axis=0, keepdims=True)

    ew = []
    ei = []
    for g in range(G):
        el = jax.lax.dot_general(werT_ref[g], xb, (((1,), (1,)), ((), ())),
                                 preferred_element_type=jnp.float32) \
            + ber_ref[g]
        mg = jnp.max(el, axis=0, keepdims=True)
        ei.append(jnp.min(jnp.where(el == mg, ridx, big), axis=0,
                          keepdims=True))
        ew.append(1.0 / jnp.sum(jnp.exp(el - mg), axis=0, keepdims=True))

    eids, ws = [], []
    for gsel, gwk in ((i1, v1), (i2, v2)):
        ew_sel = jnp.zeros((1, S), jnp.float32)
        ei_sel = jnp.zeros((1, S), jnp.int32)
        for g in range(G):
            hit = gsel == g
            ew_sel = jnp.where(hit, ew[g], ew_sel)
            ei_sel = jnp.where(hit, ei[g], ei_sel)
        eids.append(gsel * EG + ei_sel)
        ws.append(gwk * ew_sel)
    w_ref[...] = jnp.concatenate(ws, axis=0)

    e16 = jax.lax.broadcasted_iota(jnp.int32, (E, S), 0)
    o0 = (e16 == eids[0]).astype(jnp.float32)
    o1 = (e16 == eids[1]).astype(jnp.float32)
    t0 = jnp.sum(o0, axis=1, keepdims=True)
    t1 = jnp.sum(o1, axis=1, keepdims=True)
    cols = jax.lax.broadcasted_iota(jnp.int32, (S, S), 1)
    rows = jax.lax.broadcasted_iota(jnp.int32, (S, S), 0)
    stu = (rows < cols).astype(_BF)
    ob = jnp.concatenate([o0, o1], axis=0).astype(_BF)
    rex = jax.lax.dot_general(ob, stu, (((1,), (0,)), ((), ())),
                              preferred_element_type=jnp.float32)
    r0 = rex[0:E]
    r1 = rex[E:2 * E] + t0
    counts = t0 + t1
    pc = jnp.floor((counts + (BT - 1)) * (1.0 / BT)) * BT
    lt = (jax.lax.broadcasted_iota(jnp.int32, (E, E), 1)
          < jax.lax.broadcasted_iota(jnp.int32, (E, E), 0)).astype(jnp.float32)
    pstart = jax.lax.dot_general(lt, pc, (((1,), (0,)), ((), ())),
                                 preferred_element_type=jnp.float32,
                                 precision=_HI)
    p0 = jnp.sum(o0 * (pstart + r0), axis=0, keepdims=True)
    p1 = jnp.sum(o1 * (pstart + r1), axis=0, keepdims=True)
    ppos_ref[...] = jnp.concatenate([p0, p1], axis=0).astype(jnp.int32)
    meta_ref[...] = jnp.concatenate([pstart, pc], axis=1).astype(jnp.int32)


def _route(x_bf, Wg, bg, Wer, ber):
    wgT = Wg.T.astype(_BF)
    bg2 = bg.reshape(G, 1)
    werT = Wer.transpose(0, 2, 1).astype(_BF)
    ber3 = ber.reshape(G, EG, 1)
    return pl.pallas_call(
        _router_body,
        out_shape=(jax.ShapeDtypeStruct((2, S), jnp.int32),
                   jax.ShapeDtypeStruct((2, S), jnp.float32),
                   jax.ShapeDtypeStruct((E, 2), jnp.int32)),
    )(x_bf, wgT, bg2, werT, ber3)


def _gather_scatter(table, src_idx, dst_idx):
    n = src_idx.shape[0]
    dm = table.shape[1]
    win = 128
    si = src_idx.reshape(1, n)
    di = dst_idx.reshape(1, n)
    mesh = plsc.VectorSubcoreMesh(core_axis_name="c", subcore_axis_name="s")

    @functools.partial(pl.kernel,
                       out_type=jax.ShapeDtypeStruct((n, dm), table.dtype),
                       mesh=mesh,
                       scratch_types=[pltpu.VMEM((win, dm), table.dtype)])
    def k(x_hbm, si_hbm, di_hbm, o_hbm, rows_v):
        def body(si_vmem, di_vmem):
            pltpu.sync_copy(x_hbm.at[si_vmem.at[0]], rows_v)
            pltpu.sync_copy(rows_v, o_hbm.at[di_vmem.at[0]])

        pltpu.emit_pipeline(
            body,
            grid=(n // win,),
            in_specs=[pl.BlockSpec((1, win), lambda i: (0, i)),
                      pl.BlockSpec((1, win), lambda i: (0, i))],
            out_specs=[],
            core_axis_name=("c", "s"),
            dimension_semantics=(pltpu.PARALLEL,),
        )(si_hbm, di_hbm)

    return k(table, si, di)


def _ffn_body(be_ref, nu_ref, ppos_ref, x_ref, w1_ref, b1_ref, w2_ref,
              b2_ref, out_ref):
    b = pl.program_id(0)

    @pl.when(b < nu_ref[0])
    def _():
        base = b * BT
        p0 = ppos_ref[0:1, :] - base
        p1 = ppos_ref[1:2, :] - base
        slot = jax.lax.broadcasted_iota(jnp.int32, (BT, S), 0)
        oh = ((p0 == slot) | (p1 == slot)).astype(_BF)
        xv = jnp.dot(oh, x_ref[...],
                     preferred_element_type=jnp.float32).astype(_BF)
        h = jnp.dot(xv, w1_ref[0].astype(_BF),
                    preferred_element_type=jnp.float32) + b1_ref[0]
        h = jax.nn.gelu(h.astype(_BF))
        out_ref[...] = jnp.dot(h, w2_ref[0].astype(_BF),
                               preferred_element_type=jnp.float32) + b2_ref[0]


def _ffn(x_bf, ppos, block_expert, nb_used, W1, b1, W2, b2):
    w1r = W1.reshape(E, D, H)
    b1r = b1.reshape(E, 1, H)
    w2r = W2.reshape(E, H, OUTD)
    b2r = b2.reshape(E, 1, OUTD)
    grid_spec = pltpu.PrefetchScalarGridSpec(
        num_scalar_prefetch=2,
        grid=(NBCAP,),
        in_specs=[
            pl.BlockSpec((2, S), lambda b, be, nu: (0, 0)),
            pl.BlockSpec((S, D), lambda b, be, nu: (0, 0)),
            pl.BlockSpec((1, D, H), lambda b, be, nu: (be[b], 0, 0)),
            pl.BlockSpec((1, 1, H), lambda b, be, nu: (be[b], 0, 0)),
            pl.BlockSpec((1, H, OUTD), lambda b, be, nu: (be[b], 0, 0)),
            pl.BlockSpec((1, 1, OUTD), lambda b, be, nu: (be[b], 0, 0)),
        ],
        out_specs=pl.BlockSpec((BT, OUTD), lambda b, be, nu: (b, 0)),
    )
    return pl.pallas_call(
        _ffn_body,
        grid_spec=grid_spec,
        out_shape=jax.ShapeDtypeStruct((PCAP, OUTD), jnp.float32),
    )(block_expert, nb_used, ppos, x_bf, w1r, b1r, w2r, b2r)


def _combine_body(yg_ref, w0_ref, w1_ref, wo_ref, bo_ref, gam_ref, bet_ref,
                  out_ref):
    comb = (w0_ref[...] * yg_ref[0:S, :] + w1_ref[...] * yg_ref[S:2 * S, :])
    z = jnp.dot(comb.astype(_BF), wo_ref[...].astype(_BF),
                preferred_element_type=jnp.float32) + bo_ref[...]
    mu = jnp.mean(z, axis=-1, keepdims=True)
    var = jnp.mean((z - mu) ** 2, axis=-1, keepdims=True)
    out_ref[...] = (z - mu) * jax.lax.rsqrt(var + 1e-5) * gam_ref[...] \
        + bet_ref[...]


def _combine(yg, w0c, w1c, Wo, bo, gamma, beta):
    return pl.pallas_call(
        _combine_body,
        out_shape=jax.ShapeDtypeStruct((S, OUTD), jnp.float32),
    )(yg, w0c, w1c, Wo, bo.reshape(1, OUTD),
      gamma.reshape(1, OUTD), beta.reshape(1, OUTD))


def kernel(x, Wg, bg, Wer, ber, W1, b1, W2, b2, Wo, bo, gamma, beta):
    x2 = x.reshape(S, D)
    x_bf2 = x2.astype(_BF)
    ppos, w, meta = _route(x_bf2, Wg, bg, Wer, ber)

    pstart = meta[:, 0]
    pc = meta[:, 1]
    block_expert = (jnp.searchsorted(pstart // BT, jnp.arange(NBCAP),
                                     side="right") - 1).astype(jnp.int32)
    nb_used = ((pstart[E - 1] + pc[E - 1]) // BT).astype(jnp.int32).reshape(1)

    ys = _ffn(x_bf2, ppos, block_expert, nb_used, W1, b1, W2, b2)
    yg = _gather_scatter(ys, ppos.reshape(2 * S),
                         jnp.arange(2 * S, dtype=jnp.int32))
    wt = w.T
    out = _combine(yg, wt[:, 0:1], wt[:, 1:2], Wo, bo, gamma, beta)
    return out.reshape(1, S, OUTD)
